# baseline (speedup 1.0000x reference)
import functools

import jax
import jax.numpy as jnp
from jax import lax
from jax.experimental import pallas as pl
from jax.experimental.pallas import tpu as pltpu
from jax.experimental.pallas import tpu_sc as plsc


_CHUNK = 16
_NBUF = 7
_LAG = 3


_SC_ROWS = 4096
_TC_BLOCK = 512


@functools.lru_cache(maxsize=None)
def _build_copy(seq_len: int, sc_rows: int, embed_dim: int, dtype_name: str):
    dtype = jnp.dtype(dtype_name)
    info = plsc.get_sparse_core_info()
    nc, ns = info.num_cores, info.num_subcores
    nw = nc * ns
    assert sc_rows % (nw * _CHUNK) == 0
    rows_per_w = sc_rows // nw
    nchunks = rows_per_w // _CHUNK

    mesh = plsc.VectorSubcoreMesh(core_axis_name="c", subcore_axis_name="s")

    def body(table_hbm, out_hbm, *scratch):
        bufs = scratch[:_NBUF]
        in_sems = scratch[_NBUF:2 * _NBUF]
        out_sems = scratch[2 * _NBUF:3 * _NBUF]
        wid = lax.axis_index("s") * nc + lax.axis_index("c")
        base = wid * rows_per_w

        in_d = [None] * nchunks
        out_d = [None] * nchunks
        for i in range(nchunks + _LAG):
            if i < nchunks:
                b = i % _NBUF
                if i >= _NBUF:
                    out_d[i - _NBUF].wait()
                in_d[i] = pltpu.async_copy(
                    table_hbm.at[pl.ds(base + i * _CHUNK, _CHUNK)], bufs[b],
                    in_sems[b])
            if i >= _LAG:
                j = i - _LAG
                in_d[j].wait()
                out_d[j] = pltpu.async_copy(
                    bufs[j % _NBUF],
                    out_hbm.at[pl.ds(base + j * _CHUNK, _CHUNK)],
                    out_sems[j % _NBUF])
        for j in range(max(0, nchunks - _NBUF), nchunks):
            out_d[j].wait()

    return pl.kernel(
        body,
        out_type=jax.ShapeDtypeStruct((sc_rows, embed_dim), dtype),
        mesh=mesh,
        scratch_types=(
            [pltpu.VMEM((_CHUNK, embed_dim), dtype) for _ in range(_NBUF)]
            + [pltpu.SemaphoreType.DMA for _ in range(2 * _NBUF)]
        ),
    )


@functools.lru_cache(maxsize=None)
def _build_tc_copy(seq_len: int, row0: int, embed_dim: int, dtype_name: str):
    dtype = jnp.dtype(dtype_name)
    nrows = seq_len - row0
    assert nrows % _TC_BLOCK == 0 and row0 % _TC_BLOCK == 0

    def body(in_ref, out_ref):
        out_ref[...] = in_ref[...]

    return pl.pallas_call(
        body,
        grid=(nrows // _TC_BLOCK,),
        in_specs=[pl.BlockSpec((_TC_BLOCK, embed_dim),
                               lambda i: (row0 // _TC_BLOCK + i, 0))],
        out_specs=pl.BlockSpec((_TC_BLOCK, embed_dim), lambda i: (i, 0)),
        out_shape=jax.ShapeDtypeStruct((nrows, embed_dim), dtype),
    )


def kernel(idx, table):
    seq_len = idx.shape[1]
    embed_dim = table.shape[1]
    dname = table.dtype.name
    sc_part = _build_copy(seq_len, _SC_ROWS, embed_dim, dname)(table)
    tc_part = _build_tc_copy(seq_len, _SC_ROWS, embed_dim, dname)(table)
    return jnp.concatenate([sc_part, tc_part], axis=0)


# speedup vs baseline: 1.4526x; 1.4526x over previous
import functools

import jax
import jax.numpy as jnp
from jax import lax
from jax.experimental import pallas as pl
from jax.experimental.pallas import tpu as pltpu
from jax.experimental.pallas import tpu_sc as plsc


_CHUNK = 16
_NBUF = 7
_LAG = 5


_SC_ROWS = 4096
_TC_BLOCK = 512


@functools.lru_cache(maxsize=None)
def _build_copy(seq_len: int, sc_rows: int, embed_dim: int, dtype_name: str):
    dtype = jnp.dtype(dtype_name)
    info = plsc.get_sparse_core_info()
    nc, ns = info.num_cores, info.num_subcores
    nw = nc * ns
    assert sc_rows % (nw * _CHUNK) == 0
    rows_per_w = sc_rows // nw
    nchunks = rows_per_w // _CHUNK

    mesh = plsc.VectorSubcoreMesh(core_axis_name="c", subcore_axis_name="s")

    def body(table_hbm, out_hbm, *scratch):
        bufs = scratch[:_NBUF]
        in_sems = scratch[_NBUF:2 * _NBUF]
        out_sems = scratch[2 * _NBUF:3 * _NBUF]
        wid = lax.axis_index("s") * nc + lax.axis_index("c")
        base = wid * rows_per_w

        in_d = [None] * nchunks
        out_d = [None] * nchunks
        for i in range(nchunks + _LAG):
            if i < nchunks:
                b = i % _NBUF
                if i >= _NBUF:
                    out_d[i - _NBUF].wait()
                in_d[i] = pltpu.async_copy(
                    table_hbm.at[pl.ds(base + i * _CHUNK, _CHUNK)], bufs[b],
                    in_sems[b])
            if i >= _LAG:
                j = i - _LAG
                in_d[j].wait()
                out_d[j] = pltpu.async_copy(
                    bufs[j % _NBUF],
                    out_hbm.at[pl.ds(base + j * _CHUNK, _CHUNK)],
                    out_sems[j % _NBUF])
        for j in range(max(0, nchunks - _NBUF), nchunks):
            out_d[j].wait()

    return pl.kernel(
        body,
        out_type=jax.ShapeDtypeStruct((sc_rows, embed_dim), dtype),
        mesh=mesh,
        scratch_types=(
            [pltpu.VMEM((_CHUNK, embed_dim), dtype) for _ in range(_NBUF)]
            + [pltpu.SemaphoreType.DMA for _ in range(2 * _NBUF)]
        ),
    )


@functools.lru_cache(maxsize=None)
def _build_tc_copy(seq_len: int, row0: int, embed_dim: int, dtype_name: str):
    dtype = jnp.dtype(dtype_name)
    nrows = seq_len - row0
    assert nrows % _TC_BLOCK == 0 and row0 % _TC_BLOCK == 0

    def body(in_ref, out_ref):
        out_ref[...] = in_ref[...]

    return pl.pallas_call(
        body,
        grid=(nrows // _TC_BLOCK,),
        in_specs=[pl.BlockSpec((_TC_BLOCK, embed_dim),
                               lambda i: (row0 // _TC_BLOCK + i, 0))],
        out_specs=pl.BlockSpec((_TC_BLOCK, embed_dim), lambda i: (i, 0)),
        out_shape=jax.ShapeDtypeStruct((nrows, embed_dim), dtype),
    )


def kernel(idx, table):
    seq_len = idx.shape[1]
    return _build_copy(seq_len, seq_len, table.shape[1], table.dtype.name)(table)
